# baseline (speedup 1.0000x reference)
import jax
import jax.numpy as jnp
from jax.experimental import pallas as pl
from jax.experimental.pallas import tpu as pltpu

B, S, D = 1, 2048, 1024
H = 16
DH = D // H
F = 2048
E = 8
EPS = 1e-6
TS = 512
NT = S // TS
CS = 1024
FH = 2
FB = F // FH
MC = 512


def _qkv_body(x_ref, g1_ref, wq_ref, wk_ref, wv_ref, q_ref, k_ref, v_ref):
    x = x_ref[...]
    ms = jnp.mean(jnp.square(x), axis=-1, keepdims=True)
    xn = (x * jax.lax.rsqrt(ms + EPS) * g1_ref[...]).astype(jnp.bfloat16)
    q = jnp.dot(xn, wq_ref[...].astype(jnp.bfloat16),
                preferred_element_type=jnp.float32)
    q_ref[...] = (q * (1.4426950408889634 / (DH ** 0.5))).astype(jnp.bfloat16)
    k_ref[...] = jnp.dot(xn, wk_ref[...].astype(jnp.bfloat16),
                         preferred_element_type=jnp.float32).astype(jnp.bfloat16)
    v_ref[...] = jnp.dot(xn, wv_ref[...].astype(jnp.bfloat16),
                         preferred_element_type=jnp.float32).astype(jnp.bfloat16)


def _attn_body(q_ref, k_ref, v_ref, o_ref):
    ones = jnp.ones((S, 1), jnp.bfloat16)
    for h in range(H):
        hs = slice(h * DH, (h + 1) * DH)
        q = q_ref[:, hs]
        k = k_ref[:, hs]
        s = jax.lax.dot_general(q, k, (((1,), (1,)), ((), ())),
                                preferred_element_type=jnp.float32)
        m = jnp.max(s, axis=-1, keepdims=True)
        p = jnp.exp2(s - m).astype(jnp.bfloat16)
        v_ext = jnp.concatenate([v_ref[:, hs], ones], axis=1)
        o_ext = jnp.dot(p, v_ext, preferred_element_type=jnp.float32)
        o = o_ext[:, :DH] / o_ext[:, DH:DH + 1]
        o_ref[:, hs] = o.astype(jnp.bfloat16)


def _post_body(o_ref, wo_ref, inp_ref, g2_ref, wg_ref, b2_ref,
               x1_ref, xn2_ref, gate_ref):
    o = jnp.dot(o_ref[...], wo_ref[...].astype(jnp.bfloat16),
                preferred_element_type=jnp.float32)
    x1 = o + inp_ref[...]
    ms = jnp.mean(jnp.square(x1), axis=-1, keepdims=True)
    xn2 = (x1 * jax.lax.rsqrt(ms + EPS) * g2_ref[...]).astype(jnp.bfloat16)
    xn2_ref[...] = xn2
    logits = jnp.dot(xn2, wg_ref[...], preferred_element_type=jnp.float32)
    m = jnp.max(logits, axis=-1, keepdims=True)
    p = jnp.exp(logits - m)
    gate = p / jnp.sum(p, axis=-1, keepdims=True)
    gate_ref[...] = gate
    x1_ref[...] = x1 + jnp.dot(gate, b2_ref[...],
                               preferred_element_type=jnp.float32)


def _moe_body(xn_ref, x1_ref, gate_ref, w1_ref, b1_ref, w2_ref, out_ref):
    e = pl.program_id(0)
    fh = pl.program_id(1)
    first = (e == 0) & (fh == 0)
    w1b = w1_ref[0].astype(jnp.bfloat16)
    w2b = w2_ref[0].astype(jnp.bfloat16)
    b1v = b1_ref[0]
    cols = jax.lax.broadcasted_iota(jnp.int32, (MC, E), 1)
    for j in range(S // MC):
        sl = pl.ds(j * MC, MC)
        xn = xn_ref[sl, :]
        gate = gate_ref[sl, :]
        ge = jnp.sum(jnp.where(cols == e, gate, 0.0), axis=-1, keepdims=True)
        h = jnp.dot(xn, w1b, preferred_element_type=jnp.float32) + b1v
        h = (jnp.maximum(h, 0.0) * ge).astype(jnp.bfloat16)
        contrib = jnp.dot(h, w2b, preferred_element_type=jnp.float32)

        @pl.when(first)
        def _init():
            out_ref[sl, :] = x1_ref[sl, :] + contrib

        @pl.when(jnp.logical_not(first))
        def _acc():
            out_ref[sl, :] += contrib


def kernel(inputs, g1, Wq, Wk, Wv, Wo, g2, Wg, W1, b1, W2, b2):
    x = inputs.reshape(S, D)
    g1r = g1.reshape(1, D)
    g2r = g2.reshape(1, D)

    full = lambda shp: pl.BlockSpec(shp, lambda *_: tuple(0 for _ in shp))
    tok = pl.BlockSpec((TS, D), lambda t: (t, 0))

    q, k, v = pl.pallas_call(
        _qkv_body,
        grid=(NT,),
        in_specs=[tok, full((1, D)), full((D, D)), full((D, D)), full((D, D))],
        out_specs=[tok, tok, tok],
        out_shape=[jax.ShapeDtypeStruct((S, D), jnp.bfloat16)] * 3,
        compiler_params=pltpu.CompilerParams(
            dimension_semantics=("arbitrary",)),
    )(x, g1r, Wq, Wk, Wv)

    rowblk = pl.BlockSpec((CS, D), lambda t: (t, 0))
    o = pl.pallas_call(
        _attn_body,
        grid=(S // CS,),
        in_specs=[rowblk, full((S, D)), full((S, D))],
        out_specs=rowblk,
        out_shape=jax.ShapeDtypeStruct((S, D), jnp.bfloat16),
        compiler_params=pltpu.CompilerParams(
            dimension_semantics=("arbitrary",),
            vmem_limit_bytes=100 * 1024 * 1024),
    )(q, k, v)

    x1, xn2, gate = pl.pallas_call(
        _post_body,
        grid=(NT,),
        in_specs=[tok, full((D, D)), tok, full((1, D)),
                  full((D, E)), full((E, D))],
        out_specs=[tok, tok, pl.BlockSpec((TS, E), lambda t: (t, 0))],
        out_shape=[jax.ShapeDtypeStruct((S, D), jnp.float32),
                   jax.ShapeDtypeStruct((S, D), jnp.bfloat16),
                   jax.ShapeDtypeStruct((S, E), jnp.float32)],
        compiler_params=pltpu.CompilerParams(
            dimension_semantics=("arbitrary",)),
    )(o, Wo, x, g2r, Wg, b2)

    out = pl.pallas_call(
        _moe_body,
        grid=(E, FH),
        in_specs=[
            pl.BlockSpec((S, D), lambda e, f: (0, 0)),
            pl.BlockSpec((S, D), lambda e, f: (0, 0)),
            pl.BlockSpec((S, E), lambda e, f: (0, 0)),
            pl.BlockSpec((1, D, FB), lambda e, f: (e, 0, f)),
            pl.BlockSpec((1, 1, FB), lambda e, f: (e, 0, f)),
            pl.BlockSpec((1, FB, D), lambda e, f: (e, f, 0)),
        ],
        out_specs=pl.BlockSpec((S, D), lambda e, f: (0, 0)),
        out_shape=jax.ShapeDtypeStruct((S, D), jnp.float32),
        compiler_params=pltpu.CompilerParams(
            dimension_semantics=("arbitrary", "arbitrary"),
            vmem_limit_bytes=100 * 1024 * 1024),
    )(xn2, x1, gate, W1, b1.reshape(E, 1, F), W2)

    return out.reshape(B, S, D)


# speedup vs baseline: 1.0571x; 1.0571x over previous
import jax
import jax.numpy as jnp
from jax.experimental import pallas as pl
from jax.experimental.pallas import tpu as pltpu

B, S, D = 1, 2048, 1024
H = 16
DH = D // H
F = 2048
E = 8
EPS = 1e-6
TS = 512
NT = S // TS
CS = 256
FH = 2
FB = F // FH
MC = 512


def _qkv_body(x_ref, g1_ref, wq_ref, wk_ref, wv_ref, q_ref, k_ref, v_ref):
    x = x_ref[...]
    ms = jnp.mean(jnp.square(x), axis=-1, keepdims=True)
    xn = (x * jax.lax.rsqrt(ms + EPS) * g1_ref[...]).astype(jnp.bfloat16)
    q = jnp.dot(xn, wq_ref[...].astype(jnp.bfloat16),
                preferred_element_type=jnp.float32)
    q_ref[...] = (q * (1.4426950408889634 / (DH ** 0.5))).astype(jnp.bfloat16)
    k_ref[...] = jnp.dot(xn, wk_ref[...].astype(jnp.bfloat16),
                         preferred_element_type=jnp.float32).astype(jnp.bfloat16)
    v_ref[...] = jnp.dot(xn, wv_ref[...].astype(jnp.bfloat16),
                         preferred_element_type=jnp.float32).astype(jnp.bfloat16)


def _attn_body(q_ref, k_ref, v_ref, o_ref):
    ones = jnp.ones((S, 1), jnp.bfloat16)
    for h in range(H):
        hs = slice(h * DH, (h + 1) * DH)
        q = q_ref[:, hs]
        k = k_ref[:, hs]
        s = jax.lax.dot_general(q, k, (((1,), (1,)), ((), ())),
                                preferred_element_type=jnp.float32)
        m = jnp.max(s, axis=-1, keepdims=True)
        p = jnp.exp2(s - m).astype(jnp.bfloat16)
        v_ext = jnp.concatenate([v_ref[:, hs], ones], axis=1)
        o_ext = jnp.dot(p, v_ext, preferred_element_type=jnp.float32)
        o = o_ext[:, :DH] / o_ext[:, DH:DH + 1]
        o_ref[:, hs] = o.astype(jnp.bfloat16)


def _post_body(o_ref, wo_ref, inp_ref, g2_ref, wg_ref, b2_ref,
               x1_ref, xn2_ref, gate_ref):
    o = jnp.dot(o_ref[...], wo_ref[...].astype(jnp.bfloat16),
                preferred_element_type=jnp.float32)
    x1 = o + inp_ref[...]
    ms = jnp.mean(jnp.square(x1), axis=-1, keepdims=True)
    xn2 = (x1 * jax.lax.rsqrt(ms + EPS) * g2_ref[...]).astype(jnp.bfloat16)
    xn2_ref[...] = xn2
    logits = jnp.dot(xn2, wg_ref[...], preferred_element_type=jnp.float32)
    m = jnp.max(logits, axis=-1, keepdims=True)
    p = jnp.exp(logits - m)
    gate = p / jnp.sum(p, axis=-1, keepdims=True)
    gate_ref[...] = gate
    x1_ref[...] = x1 + jnp.dot(gate, b2_ref[...],
                               preferred_element_type=jnp.float32)


def _moe_body(xn_ref, x1_ref, gate_ref, w1_ref, b1_ref, w2_ref, out_ref):
    e = pl.program_id(0)
    fh = pl.program_id(1)
    first = (e == 0) & (fh == 0)
    w1b = w1_ref[0].astype(jnp.bfloat16)
    w2b = w2_ref[0].astype(jnp.bfloat16)
    b1v = b1_ref[0]
    cols = jax.lax.broadcasted_iota(jnp.int32, (MC, E), 1)
    for j in range(S // MC):
        sl = pl.ds(j * MC, MC)
        xn = xn_ref[sl, :]
        gate = gate_ref[sl, :]
        ge = jnp.sum(jnp.where(cols == e, gate, 0.0), axis=-1, keepdims=True)
        h = jnp.dot(xn, w1b, preferred_element_type=jnp.float32) + b1v
        h = (jnp.maximum(h, 0.0) * ge).astype(jnp.bfloat16)
        contrib = jnp.dot(h, w2b, preferred_element_type=jnp.float32)

        @pl.when(first)
        def _init():
            out_ref[sl, :] = x1_ref[sl, :] + contrib

        @pl.when(jnp.logical_not(first))
        def _acc():
            out_ref[sl, :] += contrib


def kernel(inputs, g1, Wq, Wk, Wv, Wo, g2, Wg, W1, b1, W2, b2):
    x = inputs.reshape(S, D)
    g1r = g1.reshape(1, D)
    g2r = g2.reshape(1, D)

    full = lambda shp: pl.BlockSpec(shp, lambda *_: tuple(0 for _ in shp))
    tok = pl.BlockSpec((TS, D), lambda t: (t, 0))

    q, k, v = pl.pallas_call(
        _qkv_body,
        grid=(NT,),
        in_specs=[tok, full((1, D)), full((D, D)), full((D, D)), full((D, D))],
        out_specs=[tok, tok, tok],
        out_shape=[jax.ShapeDtypeStruct((S, D), jnp.bfloat16)] * 3,
        compiler_params=pltpu.CompilerParams(
            dimension_semantics=("arbitrary",)),
    )(x, g1r, Wq, Wk, Wv)

    rowblk = pl.BlockSpec((CS, D), lambda t: (t, 0))
    o = pl.pallas_call(
        _attn_body,
        grid=(S // CS,),
        in_specs=[rowblk, full((S, D)), full((S, D))],
        out_specs=rowblk,
        out_shape=jax.ShapeDtypeStruct((S, D), jnp.bfloat16),
        compiler_params=pltpu.CompilerParams(
            dimension_semantics=("arbitrary",),
            vmem_limit_bytes=100 * 1024 * 1024),
    )(q, k, v)

    x1, xn2, gate = pl.pallas_call(
        _post_body,
        grid=(NT,),
        in_specs=[tok, full((D, D)), tok, full((1, D)),
                  full((D, E)), full((E, D))],
        out_specs=[tok, tok, pl.BlockSpec((TS, E), lambda t: (t, 0))],
        out_shape=[jax.ShapeDtypeStruct((S, D), jnp.float32),
                   jax.ShapeDtypeStruct((S, D), jnp.bfloat16),
                   jax.ShapeDtypeStruct((S, E), jnp.float32)],
        compiler_params=pltpu.CompilerParams(
            dimension_semantics=("arbitrary",)),
    )(o, Wo, x, g2r, Wg, b2)

    out = pl.pallas_call(
        _moe_body,
        grid=(E, FH),
        in_specs=[
            pl.BlockSpec((S, D), lambda e, f: (0, 0)),
            pl.BlockSpec((S, D), lambda e, f: (0, 0)),
            pl.BlockSpec((S, E), lambda e, f: (0, 0)),
            pl.BlockSpec((1, D, FB), lambda e, f: (e, 0, f)),
            pl.BlockSpec((1, 1, FB), lambda e, f: (e, 0, f)),
            pl.BlockSpec((1, FB, D), lambda e, f: (e, f, 0)),
        ],
        out_specs=pl.BlockSpec((S, D), lambda e, f: (0, 0)),
        out_shape=jax.ShapeDtypeStruct((S, D), jnp.float32),
        compiler_params=pltpu.CompilerParams(
            dimension_semantics=("arbitrary", "arbitrary"),
            vmem_limit_bytes=100 * 1024 * 1024),
    )(xn2, x1, gate, W1, b1.reshape(E, 1, F), W2)

    return out.reshape(B, S, D)


# speedup vs baseline: 1.1522x; 1.0900x over previous
import jax
import jax.numpy as jnp
from jax.experimental import pallas as pl
from jax.experimental.pallas import tpu as pltpu

B, S, D = 1, 2048, 1024
H = 16
DH = D // H
F = 2048
E = 8
EPS = 1e-6
TS = 512
NT = S // TS
CS = 512
FH = 2
FB = F // FH
MC = 512


def _qkv_body(x_ref, g1_ref, wq_ref, wk_ref, wv_ref, q_ref, k_ref, v_ref):
    x = x_ref[...]
    ms = jnp.mean(jnp.square(x), axis=-1, keepdims=True)
    xn = (x * jax.lax.rsqrt(ms + EPS) * g1_ref[...]).astype(jnp.bfloat16)
    q = jnp.dot(xn, wq_ref[...].astype(jnp.bfloat16),
                preferred_element_type=jnp.float32)
    q_ref[...] = (q * (1.4426950408889634 / (DH ** 0.5))).astype(jnp.bfloat16)
    k_ref[...] = jnp.dot(xn, wk_ref[...].astype(jnp.bfloat16),
                         preferred_element_type=jnp.float32).astype(jnp.bfloat16)
    v_ref[...] = jnp.dot(xn, wv_ref[...].astype(jnp.bfloat16),
                         preferred_element_type=jnp.float32).astype(jnp.bfloat16)


def _attn_body(q_ref, k_ref, v_ref, o_ref):
    ones = jnp.ones((S, 1), jnp.bfloat16)
    for h in range(H):
        hs = slice(h * DH, (h + 1) * DH)
        q = q_ref[:, hs]
        k = k_ref[:, hs]
        s = jax.lax.dot_general(q, k, (((1,), (1,)), ((), ())),
                                preferred_element_type=jnp.float32)
        m = jnp.max(s, axis=-1, keepdims=True)
        p = jnp.exp2(s - m).astype(jnp.bfloat16)
        v_ext = jnp.concatenate([v_ref[:, hs], ones], axis=1)
        o_ext = jnp.dot(p, v_ext, preferred_element_type=jnp.float32)
        o = o_ext[:, :DH] / o_ext[:, DH:DH + 1]
        o_ref[:, hs] = o.astype(jnp.bfloat16)


def _post_body(o_ref, wo_ref, inp_ref, g2_ref, wg_ref, b2_ref,
               x1_ref, xn2_ref, gate_ref):
    o = jnp.dot(o_ref[...], wo_ref[...].astype(jnp.bfloat16),
                preferred_element_type=jnp.float32)
    x1 = o + inp_ref[...]
    ms = jnp.mean(jnp.square(x1), axis=-1, keepdims=True)
    xn2 = (x1 * jax.lax.rsqrt(ms + EPS) * g2_ref[...]).astype(jnp.bfloat16)
    xn2_ref[...] = xn2
    logits = jnp.dot(xn2, wg_ref[...], preferred_element_type=jnp.float32)
    m = jnp.max(logits, axis=-1, keepdims=True)
    p = jnp.exp(logits - m)
    gate = p / jnp.sum(p, axis=-1, keepdims=True)
    gate_ref[...] = gate
    x1_ref[...] = (x1 + jnp.dot(gate, b2_ref[...],
                                preferred_element_type=jnp.float32)
                   ).astype(jnp.bfloat16)


def _moe_body(xn_ref, x1_ref, gate_ref, w1_ref, b1_ref, w2_ref, out_ref):
    e = pl.program_id(0)
    fh = pl.program_id(1)
    first = (e == 0) & (fh == 0)
    w1b = w1_ref[0].astype(jnp.bfloat16)
    w2b = w2_ref[0].astype(jnp.bfloat16)
    b1v = b1_ref[0]
    cols = jax.lax.broadcasted_iota(jnp.int32, (MC, E), 1)
    for j in range(S // MC):
        sl = pl.ds(j * MC, MC)
        xn = xn_ref[sl, :]
        gate = gate_ref[sl, :]
        ge = jnp.sum(jnp.where(cols == e, gate, 0.0), axis=-1, keepdims=True)
        h = jnp.dot(xn, w1b, preferred_element_type=jnp.float32) + b1v
        h = (jnp.maximum(h, 0.0) * ge).astype(jnp.bfloat16)
        contrib = jnp.dot(h, w2b, preferred_element_type=jnp.float32)

        @pl.when(first)
        def _init():
            out_ref[sl, :] = x1_ref[sl, :] + contrib

        @pl.when(jnp.logical_not(first))
        def _acc():
            out_ref[sl, :] += contrib


def kernel(inputs, g1, Wq, Wk, Wv, Wo, g2, Wg, W1, b1, W2, b2):
    x = inputs.reshape(S, D)
    g1r = g1.reshape(1, D)
    g2r = g2.reshape(1, D)

    full = lambda shp: pl.BlockSpec(shp, lambda *_: tuple(0 for _ in shp))
    tok = pl.BlockSpec((TS, D), lambda t: (t, 0))

    q, k, v = pl.pallas_call(
        _qkv_body,
        grid=(NT,),
        in_specs=[tok, full((1, D)), full((D, D)), full((D, D)), full((D, D))],
        out_specs=[tok, tok, tok],
        out_shape=[jax.ShapeDtypeStruct((S, D), jnp.bfloat16)] * 3,
        compiler_params=pltpu.CompilerParams(
            dimension_semantics=("arbitrary",)),
    )(x, g1r, Wq, Wk, Wv)

    rowblk = pl.BlockSpec((CS, D), lambda t: (t, 0))
    o = pl.pallas_call(
        _attn_body,
        grid=(S // CS,),
        in_specs=[rowblk, full((S, D)), full((S, D))],
        out_specs=rowblk,
        out_shape=jax.ShapeDtypeStruct((S, D), jnp.bfloat16),
        compiler_params=pltpu.CompilerParams(
            dimension_semantics=("arbitrary",),
            vmem_limit_bytes=100 * 1024 * 1024),
    )(q, k, v)

    x1, xn2, gate = pl.pallas_call(
        _post_body,
        grid=(NT,),
        in_specs=[tok, full((D, D)), tok, full((1, D)),
                  full((D, E)), full((E, D))],
        out_specs=[tok, tok, pl.BlockSpec((TS, E), lambda t: (t, 0))],
        out_shape=[jax.ShapeDtypeStruct((S, D), jnp.bfloat16),
                   jax.ShapeDtypeStruct((S, D), jnp.bfloat16),
                   jax.ShapeDtypeStruct((S, E), jnp.float32)],
        compiler_params=pltpu.CompilerParams(
            dimension_semantics=("arbitrary",)),
    )(o, Wo, x, g2r, Wg, b2)

    out = pl.pallas_call(
        _moe_body,
        grid=(E, FH),
        in_specs=[
            pl.BlockSpec((S, D), lambda e, f: (0, 0)),
            pl.BlockSpec((S, D), lambda e, f: (0, 0)),
            pl.BlockSpec((S, E), lambda e, f: (0, 0)),
            pl.BlockSpec((1, D, FB), lambda e, f: (e, 0, f)),
            pl.BlockSpec((1, 1, FB), lambda e, f: (e, 0, f)),
            pl.BlockSpec((1, FB, D), lambda e, f: (e, f, 0)),
        ],
        out_specs=pl.BlockSpec((S, D), lambda e, f: (0, 0)),
        out_shape=jax.ShapeDtypeStruct((S, D), jnp.float32),
        compiler_params=pltpu.CompilerParams(
            dimension_semantics=("arbitrary", "arbitrary"),
            vmem_limit_bytes=100 * 1024 * 1024),
    )(xn2, x1, gate, W1, b1.reshape(E, 1, F), W2)

    return out.reshape(B, S, D)


# speedup vs baseline: 1.1802x; 1.0242x over previous
import jax
import jax.numpy as jnp
from jax.experimental import pallas as pl
from jax.experimental.pallas import tpu as pltpu

B, S, D = 1, 2048, 1024
H = 16
DH = D // H
F = 2048
E = 8
EPS = 1e-6
TS = 512
NT = S // TS
CS = 512
FH = 2
FB = F // FH
MC = 1024


def _qkv_body(x_ref, g1_ref, wq_ref, wk_ref, wv_ref, q_ref, k_ref, v_ref):
    x = x_ref[...]
    ms = jnp.mean(jnp.square(x), axis=-1, keepdims=True)
    xn = (x * jax.lax.rsqrt(ms + EPS) * g1_ref[...]).astype(jnp.bfloat16)
    q = jnp.dot(xn, wq_ref[...].astype(jnp.bfloat16),
                preferred_element_type=jnp.float32)
    q_ref[...] = (q * (1.4426950408889634 / (DH ** 0.5))).astype(jnp.bfloat16)
    k_ref[...] = jnp.dot(xn, wk_ref[...].astype(jnp.bfloat16),
                         preferred_element_type=jnp.float32).astype(jnp.bfloat16)
    v_ref[...] = jnp.dot(xn, wv_ref[...].astype(jnp.bfloat16),
                         preferred_element_type=jnp.float32).astype(jnp.bfloat16)


def _attn_body(q_ref, k_ref, v_ref, o_ref):
    ones = jnp.ones((S, 1), jnp.bfloat16)
    for h in range(H):
        hs = slice(h * DH, (h + 1) * DH)
        q = q_ref[:, hs]
        k = k_ref[:, hs]
        s = jax.lax.dot_general(q, k, (((1,), (1,)), ((), ())),
                                preferred_element_type=jnp.float32)
        m = jnp.max(s, axis=-1, keepdims=True)
        p = jnp.exp2(s - m).astype(jnp.bfloat16)
        v_ext = jnp.concatenate([v_ref[:, hs], ones], axis=1)
        o_ext = jnp.dot(p, v_ext, preferred_element_type=jnp.float32)
        o = o_ext[:, :DH] / o_ext[:, DH:DH + 1]
        o_ref[:, hs] = o.astype(jnp.bfloat16)


def _post_body(o_ref, wo_ref, inp_ref, g2_ref, wg_ref, b2_ref,
               x1_ref, xn2_ref, gate_ref):
    o = jnp.dot(o_ref[...], wo_ref[...].astype(jnp.bfloat16),
                preferred_element_type=jnp.float32)
    x1 = o + inp_ref[...]
    ms = jnp.mean(jnp.square(x1), axis=-1, keepdims=True)
    xn2 = (x1 * jax.lax.rsqrt(ms + EPS) * g2_ref[...]).astype(jnp.bfloat16)
    xn2_ref[...] = xn2
    logits = jnp.dot(xn2, wg_ref[...], preferred_element_type=jnp.float32)
    m = jnp.max(logits, axis=-1, keepdims=True)
    p = jnp.exp(logits - m)
    gate = p / jnp.sum(p, axis=-1, keepdims=True)
    gate_ref[...] = gate
    x1_ref[...] = (x1 + jnp.dot(gate, b2_ref[...],
                                preferred_element_type=jnp.float32)
                   ).astype(jnp.bfloat16)


def _moe_body(xn_ref, x1_ref, gate_ref, w1_ref, b1_ref, w2_ref, out_ref):
    e = pl.program_id(0)
    fh = pl.program_id(1)
    first = (e == 0) & (fh == 0)
    w1b = w1_ref[0].astype(jnp.bfloat16)
    w2b = w2_ref[0].astype(jnp.bfloat16)
    b1v = b1_ref[0]
    cols = jax.lax.broadcasted_iota(jnp.int32, (MC, E), 1)
    for j in range(S // MC):
        sl = pl.ds(j * MC, MC)
        xn = xn_ref[sl, :]
        gate = gate_ref[sl, :]
        ge = jnp.sum(jnp.where(cols == e, gate, 0.0), axis=-1, keepdims=True)
        h = jnp.dot(xn, w1b, preferred_element_type=jnp.float32) + b1v
        h = (jnp.maximum(h, 0.0) * ge).astype(jnp.bfloat16)
        contrib = jnp.dot(h, w2b, preferred_element_type=jnp.float32)

        @pl.when(first)
        def _init():
            out_ref[sl, :] = x1_ref[sl, :] + contrib

        @pl.when(jnp.logical_not(first))
        def _acc():
            out_ref[sl, :] += contrib


def kernel(inputs, g1, Wq, Wk, Wv, Wo, g2, Wg, W1, b1, W2, b2):
    x = inputs.reshape(S, D)
    g1r = g1.reshape(1, D)
    g2r = g2.reshape(1, D)

    full = lambda shp: pl.BlockSpec(shp, lambda *_: tuple(0 for _ in shp))
    tok = pl.BlockSpec((TS, D), lambda t: (t, 0))

    q, k, v = pl.pallas_call(
        _qkv_body,
        grid=(NT,),
        in_specs=[tok, full((1, D)), full((D, D)), full((D, D)), full((D, D))],
        out_specs=[tok, tok, tok],
        out_shape=[jax.ShapeDtypeStruct((S, D), jnp.bfloat16)] * 3,
        compiler_params=pltpu.CompilerParams(
            dimension_semantics=("arbitrary",)),
    )(x, g1r, Wq, Wk, Wv)

    rowblk = pl.BlockSpec((CS, D), lambda t: (t, 0))
    o = pl.pallas_call(
        _attn_body,
        grid=(S // CS,),
        in_specs=[rowblk, full((S, D)), full((S, D))],
        out_specs=rowblk,
        out_shape=jax.ShapeDtypeStruct((S, D), jnp.bfloat16),
        compiler_params=pltpu.CompilerParams(
            dimension_semantics=("arbitrary",),
            vmem_limit_bytes=100 * 1024 * 1024),
    )(q, k, v)

    x1, xn2, gate = pl.pallas_call(
        _post_body,
        grid=(NT,),
        in_specs=[tok, full((D, D)), tok, full((1, D)),
                  full((D, E)), full((E, D))],
        out_specs=[tok, tok, pl.BlockSpec((TS, E), lambda t: (t, 0))],
        out_shape=[jax.ShapeDtypeStruct((S, D), jnp.bfloat16),
                   jax.ShapeDtypeStruct((S, D), jnp.bfloat16),
                   jax.ShapeDtypeStruct((S, E), jnp.float32)],
        compiler_params=pltpu.CompilerParams(
            dimension_semantics=("arbitrary",)),
    )(o, Wo, x, g2r, Wg, b2)

    out = pl.pallas_call(
        _moe_body,
        grid=(E, FH),
        in_specs=[
            pl.BlockSpec((S, D), lambda e, f: (0, 0)),
            pl.BlockSpec((S, D), lambda e, f: (0, 0)),
            pl.BlockSpec((S, E), lambda e, f: (0, 0)),
            pl.BlockSpec((1, D, FB), lambda e, f: (e, 0, f)),
            pl.BlockSpec((1, 1, FB), lambda e, f: (e, 0, f)),
            pl.BlockSpec((1, FB, D), lambda e, f: (e, f, 0)),
        ],
        out_specs=pl.BlockSpec((S, D), lambda e, f: (0, 0)),
        out_shape=jax.ShapeDtypeStruct((S, D), jnp.float32),
        compiler_params=pltpu.CompilerParams(
            dimension_semantics=("arbitrary", "arbitrary"),
            vmem_limit_bytes=100 * 1024 * 1024),
    )(xn2, x1, gate, W1, b1.reshape(E, 1, F), W2)

    return out.reshape(B, S, D)
